# Initial kernel scaffold; baseline (speedup 1.0000x reference)
import jax
import jax.numpy as jnp
from jax.experimental import pallas as pl


def kernel(x, labels, signatures):
    raise NotImplementedError("write your pallas kernel here")


# speedup vs baseline: 14.6289x; 14.6289x over previous
import functools
import math

import jax
import jax.numpy as jnp
import numpy as np
from jax import lax
from jax.experimental import pallas as pl
from jax.experimental.pallas import tpu as pltpu
from jax.experimental.pallas import tpu_sc as plsc

_HI = jax.lax.Precision.HIGHEST
_LANES = 16


def _dot(a, b, precision=_HI):
    return jax.lax.dot_general(a, b, (((1,), (0,)), ((), ())),
                               precision=precision,
                               preferred_element_type=jnp.float32)


def _row_normalize(v, eps=1e-12):
    nrm = jnp.sqrt(jnp.sum(v * v, axis=1, keepdims=True))
    return v / jnp.maximum(nrm, eps)


@functools.lru_cache(maxsize=4)
def _dft_basis(d):
    mf = np.outer(np.arange(d), np.arange(d)) % d
    theta = mf.astype(np.float64) * (2.0 * math.pi / d)
    return (jnp.asarray(np.cos(theta), jnp.float32),
            jnp.asarray(np.sin(theta), jnp.float32))


def _prep_body(lab_ref, sig_ref, cos_ref, sin_ref, c_ref, en_ref):
    d = lab_ref.shape[1]
    ln = _row_normalize(lab_ref[...])
    en = _row_normalize(sig_ref[...])
    en_ref[...] = en

    cosm = cos_ref[...]
    sinm = sin_ref[...]

    a = _dot(en, cosm)
    b = _dot(en, sinm)
    cc = _dot(ln, cosm)
    dd = _dot(ln, sinm)
    rr = jnp.sum(a * cc - b * dd, axis=0, keepdims=True)
    ri = -jnp.sum(a * dd + b * cc, axis=0, keepdims=True)
    t1 = (cosm * rr - sinm * ri) * (1.0 / d)
    t2 = (sinm * rr + cosm * ri) * (1.0 / d)
    c_ref[...] = _dot(t1, cosm) + _dot(t2, sinm)


def _main_body(x_ref, c_ref, en_ref, sc_ref, sct_ref):
    xn = _row_normalize(x_ref[...])
    s = _dot(xn, c_ref[...])
    sn = _row_normalize(s)
    scores = jax.lax.dot_general(sn.astype(jnp.bfloat16),
                                 en_ref[...].astype(jnp.bfloat16),
                                 (((1,), (1,)), ((), ())),
                                 preferred_element_type=jnp.float32)
    sc_ref[...] = scores
    sct_ref[...] = scores.T


def _sc_route_body(n_exp, chunk, sct_hbm, w1_hbm, w2_hbm, i1_hbm, i2_hbm,
                   sct_v, w1_v, w2_v, i1_v, i2_v):
    wid = lax.axis_index("s") * 2 + lax.axis_index("c")
    base = wid * chunk
    pltpu.sync_copy(sct_hbm.at[:, pl.ds(base, chunk)], sct_v)

    def block(b, carry_unused):
        o = b * _LANES
        neg = jnp.full((_LANES,), -jnp.inf, jnp.float32)
        zero = jnp.zeros((_LANES,), jnp.int32)
        b1, i1, b2, i2 = neg, zero, neg, zero
        for e in range(n_exp):
            v = sct_v[e, pl.ds(o, _LANES)]
            ev = zero + e
            gt1 = v > b1
            nb1 = jnp.where(gt1, v, b1)
            ni1 = jnp.where(gt1, ev, i1)
            dem = jnp.where(gt1, b1, v)
            demi = jnp.where(gt1, i1, ev)
            gt2 = dem > b2
            b2 = jnp.where(gt2, dem, b2)
            i2 = jnp.where(gt2, demi, i2)
            b1, i1 = nb1, ni1
        e2 = jnp.exp(b2 - b1)
        den = 1.0 + e2
        w1_v[pl.ds(o, _LANES)] = 1.0 / den
        w2_v[pl.ds(o, _LANES)] = e2 / den
        i1_v[pl.ds(o, _LANES)] = i1
        i2_v[pl.ds(o, _LANES)] = i2
        return carry_unused

    lax.fori_loop(0, chunk // _LANES, block, 0)
    pltpu.sync_copy(w1_v, w1_hbm.at[pl.ds(base, chunk)])
    pltpu.sync_copy(w2_v, w2_hbm.at[pl.ds(base, chunk)])
    pltpu.sync_copy(i1_v, i1_hbm.at[pl.ds(base, chunk)])
    pltpu.sync_copy(i2_v, i2_hbm.at[pl.ds(base, chunk)])


def _sc_route(scores_t):
    e, n = scores_t.shape
    chunk = n // 32
    mesh = plsc.VectorSubcoreMesh(core_axis_name="c", subcore_axis_name="s")
    f32, i32 = jnp.float32, jnp.int32
    return pl.kernel(
        functools.partial(_sc_route_body, e, chunk),
        mesh=mesh,
        out_type=[
            jax.ShapeDtypeStruct((n,), f32),
            jax.ShapeDtypeStruct((n,), f32),
            jax.ShapeDtypeStruct((n,), i32),
            jax.ShapeDtypeStruct((n,), i32),
        ],
        scratch_types=[
            pltpu.VMEM((e, chunk), f32),
            pltpu.VMEM((chunk,), f32),
            pltpu.VMEM((chunk,), f32),
            pltpu.VMEM((chunk,), i32),
            pltpu.VMEM((chunk,), i32),
        ],
    )(scores_t)


def kernel(x, labels, signatures):
    n, d = x.shape
    e = labels.shape[0]

    cosm, sinm = _dft_basis(d)
    c_mat, en = pl.pallas_call(
        _prep_body,
        out_shape=[
            jax.ShapeDtypeStruct((d, d), jnp.float32),
            jax.ShapeDtypeStruct((e, d), jnp.float32),
        ],
    )(labels, signatures, cosm, sinm)

    tile = 1024
    while n % tile:
        tile //= 2
    grid = (n // tile,)

    scores, scores_t = pl.pallas_call(
        _main_body,
        grid=grid,
        in_specs=[
            pl.BlockSpec((tile, d), lambda i: (i, 0)),
            pl.BlockSpec((d, d), lambda i: (0, 0)),
            pl.BlockSpec((e, d), lambda i: (0, 0)),
        ],
        out_specs=[
            pl.BlockSpec((tile, e), lambda i: (i, 0)),
            pl.BlockSpec((e, tile), lambda i: (0, i)),
        ],
        out_shape=[
            jax.ShapeDtypeStruct((n, e), jnp.float32),
            jax.ShapeDtypeStruct((e, n), jnp.float32),
        ],
    )(x, c_mat, en)

    w1, w2, i1, i2 = _sc_route(scores_t)
    weights = jnp.stack([w1, w2], axis=1)
    indices = jnp.stack([i1, i2], axis=1)
    return weights, indices, scores
